# baseline (speedup 1.0000x reference)
import functools

import jax
import jax.numpy as jnp
from jax import lax
from jax.experimental import pallas as pl
from jax.experimental.pallas import tpu as pltpu
from jax.experimental.pallas import tpu_sc as plsc

N = 10000
E = 320000
D_IN = 128
D_H = 128
N_CLASSES = 40

NC = 2
NS = 16
NW = NC * NS
EPW = E // NW
RPW = N // NS

D0 = 144
D1 = 128
D2 = 48

ROW_BLK = 2000
GRID = N // ROW_BLK


NBUF = 5
ILEAD = 3
GLEAD = 2
SLAG = 2
_CHUNK_BY_D = {144: 40, 128: 40, 48: 40}


def _make_sc_agg(d):
  chunk = _CHUNK_BY_D[d]
  nchunk = EPW // chunk
  mesh = plsc.VectorSubcoreMesh(core_axis_name="c", subcore_axis_name="s",
                                num_cores=NC, num_subcores=NS)

  @functools.partial(
      pl.kernel,
      out_type=jax.ShapeDtypeStruct((NC, N, d), jnp.float32),
      mesh=mesh,
      scratch_types=[
          [pltpu.VMEM((2, chunk), jnp.int32) for _ in range(NBUF)],
          [pltpu.VMEM((chunk, d), jnp.float32) for _ in range(NBUF)],
          pltpu.SemaphoreType.DMA((NBUF,)),
          pltpu.SemaphoreType.DMA((NBUF,)),
          pltpu.SemaphoreType.DMA((NBUF,)),
          pltpu.VMEM_SHARED((N, d), jnp.float32),
      ],
      compiler_params=pltpu.CompilerParams(use_tc_tiling_on_sc=False),
  )
  def agg(table, edges, zeros, out, ibufs, rbufs, isem, gsem, ssem, acc):
    cid = lax.axis_index("c")
    sid = lax.axis_index("s")
    wid = sid * NC + cid

    pltpu.sync_copy(zeros, acc.at[pl.ds(sid * RPW, RPW)])
    plsc.subcore_barrier()

    base = wid * EPW

    def fire_idx(c, b):
      pltpu.async_copy(edges.at[:, pl.ds(base + c * chunk, chunk)], ibufs[b],
                       isem.at[b])

    def fire_gather(c, b):
      pltpu.make_async_copy(edges.at[:, pl.ds(base + c * chunk, chunk)],
                            ibufs[b], isem.at[b]).wait()
      pltpu.async_copy(table.at[ibufs[b].at[0]], rbufs[b], gsem.at[b])

    for c in range(ILEAD):
      fire_idx(c, c)
    for c in range(GLEAD):
      fire_gather(c, c)

    def grp(g, carry):
      for k in range(NBUF):
        c = g * NBUF + k
        bs = (k + NBUF - SLAG) % NBUF
        bg = (k + GLEAD) % NBUF
        @pl.when(c >= SLAG)
        def _():
          pltpu.make_async_copy(rbufs[bs], acc.at[ibufs[bs].at[1]],
                                ssem.at[bs]).wait()
        @pl.when(c + ILEAD < nchunk)
        def _():
          fire_idx(c + ILEAD, (k + ILEAD) % NBUF)
        @pl.when(c + GLEAD < nchunk)
        def _():
          fire_gather(c + GLEAD, bg)
        pltpu.make_async_copy(table.at[ibufs[k].at[0]], rbufs[k],
                              gsem.at[k]).wait()
        pltpu.async_copy(rbufs[k], acc.at[ibufs[k].at[1]], ssem.at[k],
                         add=True)
      return carry

    lax.fori_loop(0, nchunk // NBUF, grp, 0)
    for c in range(nchunk - SLAG, nchunk):
      b = c % NBUF
      pltpu.make_async_copy(rbufs[b], acc.at[ibufs[b].at[1]],
                            ssem.at[b]).wait()
    plsc.subcore_barrier()

    pltpu.sync_copy(acc.at[pl.ds(sid * RPW, RPW)],
                    out.at[cid, pl.ds(sid * RPW, RPW)])

  return agg


_sc_agg_cache = {}


def _agg(table, edges, zeros, d):
  if d not in _sc_agg_cache:
    _sc_agg_cache[d] = _make_sc_agg(d)
  return _sc_agg_cache[d](table, edges, zeros)


def _p_specs(d):
  return [
      pl.BlockSpec((1, ROW_BLK, d), lambda i: (0, i, 0)),
      pl.BlockSpec((1, ROW_BLK, d), lambda i: (1, i, 0)),
  ]


def _stage1_body(x_ref, w_ref, b_ref, o_ref):
  o_ref[...] = (jnp.dot(x_ref[...], w_ref[...],
                        preferred_element_type=jnp.float32) + b_ref[...])


def _tc_stage1(x, w, b):
  return pl.pallas_call(
      _stage1_body,
      grid=(GRID,),
      in_specs=[
          pl.BlockSpec((ROW_BLK, D_IN), lambda i: (i, 0)),
          pl.BlockSpec((D_IN, D0), lambda i: (0, 0)),
          pl.BlockSpec((1, D0), lambda i: (0, 0)),
      ],
      out_specs=pl.BlockSpec((ROW_BLK, D0), lambda i: (i, 0)),
      out_shape=jax.ShapeDtypeStruct((N, D0), jnp.float32),
  )(x, w, b)


def _stage2_body(p0_ref, p1_ref, b_ref, w_ref, t_ref, dinv_ref):
  s = p0_ref[0] + p1_ref[0]
  deg = s[:, D_H:D_H + 1]
  dinv = 1.0 / jnp.maximum(deg, 1.0)
  h = jnp.maximum(s[:, :D_H] * dinv + b_ref[...], 0.0)
  t_ref[...] = jnp.dot(h, w_ref[...], preferred_element_type=jnp.float32)
  dinv_ref[...] = dinv


def _tc_stage2(p, b0, w1):
  return pl.pallas_call(
      _stage2_body,
      grid=(GRID,),
      in_specs=_p_specs(D0) + [
          pl.BlockSpec((1, D_H), lambda i: (0, 0)),
          pl.BlockSpec((D_H, D_H), lambda i: (0, 0)),
      ],
      out_specs=[
          pl.BlockSpec((ROW_BLK, D_H), lambda i: (i, 0)),
          pl.BlockSpec((ROW_BLK, 1), lambda i: (i, 0)),
      ],
      out_shape=[
          jax.ShapeDtypeStruct((N, D_H), jnp.float32),
          jax.ShapeDtypeStruct((N, 1), jnp.float32),
      ],
  )(p, p, b0, w1)


def _stage3_body(p0_ref, p1_ref, dinv_ref, b_ref, w_ref, t_ref):
  s = p0_ref[0] + p1_ref[0]
  h = jnp.maximum(s * dinv_ref[...] + b_ref[...], 0.0)
  t_ref[...] = jnp.dot(h, w_ref[...], preferred_element_type=jnp.float32)


def _tc_stage3(p, dinv, b1, w2):
  return pl.pallas_call(
      _stage3_body,
      grid=(GRID,),
      in_specs=_p_specs(D1) + [
          pl.BlockSpec((ROW_BLK, 1), lambda i: (i, 0)),
          pl.BlockSpec((1, D_H), lambda i: (0, 0)),
          pl.BlockSpec((D_H, D2), lambda i: (0, 0)),
      ],
      out_specs=pl.BlockSpec((ROW_BLK, D2), lambda i: (i, 0)),
      out_shape=jax.ShapeDtypeStruct((N, D2), jnp.float32),
  )(p, p, dinv, b1, w2)


def _stage4_body(p0_ref, p1_ref, dinv_ref, b_ref, o_ref):
  s = p0_ref[0, :, :N_CLASSES] + p1_ref[0, :, :N_CLASSES]
  o_ref[...] = s * dinv_ref[...] + b_ref[...]


def _tc_stage4(p, dinv, b2):
  return pl.pallas_call(
      _stage4_body,
      grid=(GRID,),
      in_specs=_p_specs(D2) + [
          pl.BlockSpec((ROW_BLK, 1), lambda i: (i, 0)),
          pl.BlockSpec((1, N_CLASSES), lambda i: (0, 0)),
      ],
      out_specs=pl.BlockSpec((ROW_BLK, N_CLASSES), lambda i: (i, 0)),
      out_shape=jax.ShapeDtypeStruct((N, N_CLASSES), jnp.float32),
  )(p, p, dinv, b2)


def kernel(features, edge_index, W0, b0, W1, b1, W2, b2):
  ei = edge_index.astype(jnp.int32)

  w0pad = jnp.zeros((D_IN, D0), jnp.float32).at[:, :D_H].set(W0)
  b0aug = jnp.zeros((1, D0), jnp.float32).at[0, D_H].set(1.0)

  w2pad = jnp.zeros((D_H, D2), jnp.float32).at[:, :N_CLASSES].set(W2)

  z0 = jnp.zeros((RPW, D0), jnp.float32)
  z1 = jnp.zeros((RPW, D1), jnp.float32)
  z2 = jnp.zeros((RPW, D2), jnp.float32)

  t0 = _tc_stage1(features, w0pad, b0aug)
  p = _agg(t0, ei, z0, D0)
  t1, dinv = _tc_stage2(p, b0.reshape(1, D_H), W1)
  p = _agg(t1, ei, z1, D1)
  t2 = _tc_stage3(p, dinv, b1.reshape(1, D_H), w2pad)
  p = _agg(t2, ei, z2, D2)
  return _tc_stage4(p, dinv, b2.reshape(1, N_CLASSES))


# speedup vs baseline: 1.1220x; 1.1220x over previous
import functools

import jax
import jax.numpy as jnp
from jax import lax
from jax.experimental import pallas as pl
from jax.experimental.pallas import tpu as pltpu
from jax.experimental.pallas import tpu_sc as plsc

N = 10000
E = 320000
D_IN = 128
D_H = 128
N_CLASSES = 40

NC = 2
NS = 16
NW = NC * NS
EPW = E // NW
RPW = N // NS

D0 = 136
D1 = 128
D2 = 48

ROW_BLK = 2000
GRID = N // ROW_BLK


CHUNK = 80
SLAG = 2
_NBUF_BY_D = {136: 4, 128: 4, 48: 5}


def _make_sc_agg(d):
  chunk = CHUNK
  nchunk = EPW // chunk
  nbuf = _NBUF_BY_D[d]
  ilead = nbuf - SLAG
  glead = ilead - 1
  ngrp = nchunk // nbuf
  nmain = ngrp * nbuf
  mesh = plsc.VectorSubcoreMesh(core_axis_name="c", subcore_axis_name="s",
                                num_cores=NC, num_subcores=NS)

  @functools.partial(
      pl.kernel,
      out_type=jax.ShapeDtypeStruct((NC, N, d), jnp.float32),
      mesh=mesh,
      scratch_types=[
          [pltpu.VMEM((2, chunk), jnp.int32) for _ in range(nbuf)],
          [pltpu.VMEM((chunk, d), jnp.float32) for _ in range(nbuf)],
          pltpu.SemaphoreType.DMA((nbuf,)),
          pltpu.SemaphoreType.DMA((nbuf,)),
          pltpu.SemaphoreType.DMA((nbuf,)),
          pltpu.VMEM_SHARED((N, d), jnp.float32),
      ],
      compiler_params=pltpu.CompilerParams(use_tc_tiling_on_sc=False),
  )
  def agg(table, edges, zeros, out, ibufs, rbufs, isem, gsem, ssem, acc):
    cid = lax.axis_index("c")
    sid = lax.axis_index("s")
    wid = sid * NC + cid

    pltpu.sync_copy(zeros, acc.at[pl.ds(sid * RPW, RPW)])
    plsc.subcore_barrier()

    base = wid * EPW

    def fire_idx(c, b):
      pltpu.async_copy(edges.at[:, pl.ds(base + c * chunk, chunk)], ibufs[b],
                       isem.at[b])

    def fire_gather(c, b):
      pltpu.make_async_copy(edges.at[:, pl.ds(base + c * chunk, chunk)],
                            ibufs[b], isem.at[b]).wait()
      pltpu.async_copy(table.at[ibufs[b].at[0]], rbufs[b], gsem.at[b])

    def drain_scatter(b):
      pltpu.make_async_copy(rbufs[b], acc.at[ibufs[b].at[1]],
                            ssem.at[b]).wait()

    def consume(c, b):
      pltpu.make_async_copy(table.at[ibufs[b].at[0]], rbufs[b],
                            gsem.at[b]).wait()
      pltpu.async_copy(rbufs[b], acc.at[ibufs[b].at[1]], ssem.at[b],
                       add=True)

    for c in range(ilead):
      fire_idx(c, c)
    for c in range(glead):
      fire_gather(c, c)

    def make_grp(guarded):
      def grp(g, carry):
        for k in range(nbuf):
          c = g * nbuf + k
          bs = (k + nbuf - SLAG) % nbuf
          bg = (k + glead) % nbuf
          if guarded:
            if c >= SLAG:
              drain_scatter(bs)
            if c + ilead < nchunk:
              fire_idx(c + ilead, (k + ilead) % nbuf)
            if c + glead < nchunk:
              fire_gather(c + glead, bg)
          else:
            drain_scatter(bs)
            fire_idx(c + ilead, (k + ilead) % nbuf)
            fire_gather(c + glead, bg)
          consume(c, k)
        return carry
      return grp

    make_grp(True)(0, 0)
    lax.fori_loop(1, ngrp - 1, make_grp(False), 0)
    make_grp(True)(ngrp - 1, 0)
    for c in range(nmain, nchunk):
      pltpu.make_async_copy(table.at[ibufs[c % nbuf].at[0]], rbufs[c % nbuf],
                            gsem.at[c % nbuf]).wait()
      pltpu.async_copy(rbufs[c % nbuf], acc.at[ibufs[c % nbuf].at[1]],
                       ssem.at[c % nbuf], add=True)
    for c in range(nmain - SLAG, nchunk):
      drain_scatter(c % nbuf)
    plsc.subcore_barrier()

    pltpu.sync_copy(acc.at[pl.ds(sid * RPW, RPW)],
                    out.at[cid, pl.ds(sid * RPW, RPW)])

  return agg


_sc_agg_cache = {}


def _agg(table, edges, zeros, d):
  if d not in _sc_agg_cache:
    _sc_agg_cache[d] = _make_sc_agg(d)
  return _sc_agg_cache[d](table, edges, zeros)


def _p_specs(d):
  return [
      pl.BlockSpec((1, ROW_BLK, d), lambda i: (0, i, 0)),
      pl.BlockSpec((1, ROW_BLK, d), lambda i: (1, i, 0)),
  ]


def _stage1_body(x_ref, w_ref, b_ref, o_ref):
  o_ref[...] = (jnp.dot(x_ref[...], w_ref[...],
                        preferred_element_type=jnp.float32) + b_ref[...])


def _tc_stage1(x, w, b):
  return pl.pallas_call(
      _stage1_body,
      grid=(GRID,),
      in_specs=[
          pl.BlockSpec((ROW_BLK, D_IN), lambda i: (i, 0)),
          pl.BlockSpec((D_IN, D0), lambda i: (0, 0)),
          pl.BlockSpec((1, D0), lambda i: (0, 0)),
      ],
      out_specs=pl.BlockSpec((ROW_BLK, D0), lambda i: (i, 0)),
      out_shape=jax.ShapeDtypeStruct((N, D0), jnp.float32),
  )(x, w, b)


def _stage2_body(p0_ref, p1_ref, b_ref, w_ref, t_ref, dinv_ref):
  s = p0_ref[0] + p1_ref[0]
  deg = s[:, D_H:D_H + 1]
  dinv = 1.0 / jnp.maximum(deg, 1.0)
  h = jnp.maximum(s[:, :D_H] * dinv + b_ref[...], 0.0)
  t_ref[...] = jnp.dot(h, w_ref[...], preferred_element_type=jnp.float32)
  dinv_ref[...] = dinv


def _tc_stage2(p, b0, w1):
  return pl.pallas_call(
      _stage2_body,
      grid=(GRID,),
      in_specs=_p_specs(D0) + [
          pl.BlockSpec((1, D_H), lambda i: (0, 0)),
          pl.BlockSpec((D_H, D_H), lambda i: (0, 0)),
      ],
      out_specs=[
          pl.BlockSpec((ROW_BLK, D_H), lambda i: (i, 0)),
          pl.BlockSpec((ROW_BLK, 1), lambda i: (i, 0)),
      ],
      out_shape=[
          jax.ShapeDtypeStruct((N, D_H), jnp.float32),
          jax.ShapeDtypeStruct((N, 1), jnp.float32),
      ],
  )(p, p, b0, w1)


def _stage3_body(p0_ref, p1_ref, dinv_ref, b_ref, w_ref, t_ref):
  s = p0_ref[0] + p1_ref[0]
  h = jnp.maximum(s * dinv_ref[...] + b_ref[...], 0.0)
  t_ref[...] = jnp.dot(h, w_ref[...], preferred_element_type=jnp.float32)


def _tc_stage3(p, dinv, b1, w2):
  return pl.pallas_call(
      _stage3_body,
      grid=(GRID,),
      in_specs=_p_specs(D1) + [
          pl.BlockSpec((ROW_BLK, 1), lambda i: (i, 0)),
          pl.BlockSpec((1, D_H), lambda i: (0, 0)),
          pl.BlockSpec((D_H, D2), lambda i: (0, 0)),
      ],
      out_specs=pl.BlockSpec((ROW_BLK, D2), lambda i: (i, 0)),
      out_shape=jax.ShapeDtypeStruct((N, D2), jnp.float32),
  )(p, p, dinv, b1, w2)


def _stage4_body(p0_ref, p1_ref, dinv_ref, b_ref, o_ref):
  s = p0_ref[0, :, :N_CLASSES] + p1_ref[0, :, :N_CLASSES]
  o_ref[...] = s * dinv_ref[...] + b_ref[...]


def _tc_stage4(p, dinv, b2):
  return pl.pallas_call(
      _stage4_body,
      grid=(GRID,),
      in_specs=_p_specs(D2) + [
          pl.BlockSpec((ROW_BLK, 1), lambda i: (i, 0)),
          pl.BlockSpec((1, N_CLASSES), lambda i: (0, 0)),
      ],
      out_specs=pl.BlockSpec((ROW_BLK, N_CLASSES), lambda i: (i, 0)),
      out_shape=jax.ShapeDtypeStruct((N, N_CLASSES), jnp.float32),
  )(p, p, dinv, b2)


def kernel(features, edge_index, W0, b0, W1, b1, W2, b2):
  ei = edge_index.astype(jnp.int32)

  w0pad = jnp.zeros((D_IN, D0), jnp.float32).at[:, :D_H].set(W0)
  b0aug = jnp.zeros((1, D0), jnp.float32).at[0, D_H].set(1.0)

  w2pad = jnp.zeros((D_H, D2), jnp.float32).at[:, :N_CLASSES].set(W2)

  z0 = jnp.zeros((RPW, D0), jnp.float32)
  z1 = jnp.zeros((RPW, D1), jnp.float32)
  z2 = jnp.zeros((RPW, D2), jnp.float32)

  t0 = _tc_stage1(features, w0pad, b0aug)
  p = _agg(t0, ei, z0, D0)
  t1, dinv = _tc_stage2(p, b0.reshape(1, D_H), W1)
  p = _agg(t1, ei, z1, D1)
  t2 = _tc_stage3(p, dinv, b1.reshape(1, D_H), w2pad)
  p = _agg(t2, ei, z2, D2)
  return _tc_stage4(p, dinv, b2.reshape(1, N_CLASSES))
